# baseline (speedup 1.0000x reference)
import functools

import jax
import jax.numpy as jnp
from jax import lax
from jax.experimental import pallas as pl
from jax.experimental.pallas import tpu as pltpu
from jax.experimental.pallas import tpu_sc as plsc

K = 32
LANE_DEPTH = 5


def _f32_key(x):
    bits = lax.bitcast_convert_type(x, jnp.uint32)
    flip = jnp.where(
        (bits >> jnp.uint32(31)) > jnp.uint32(0),
        jnp.uint32(0xFFFFFFFF),
        jnp.uint32(0x80000000),
    )
    return bits ^ flip


def _key_to_f32(k):
    pos = (k >> jnp.uint32(31)) > jnp.uint32(0)
    bits = jnp.where(pos, k ^ jnp.uint32(0x80000000), ~k)
    return lax.bitcast_convert_type(bits, jnp.float32)


def _tc_body(x_ref, we_ref, be_ref, lat_ref, cval_ref, cidx_ref,
             pre_ref, cv_ref, ci_ref, tval_ref, *, nj):
    j = pl.program_id(1)
    r = x_ref.shape[0]
    sae_blk = we_ref.shape[0]
    nseg = sae_blk // 128
    neg = jnp.float32(-jnp.inf)

    @pl.when(j < nj)
    def _encode():
        acc = lax.dot_general(
            x_ref[...], we_ref[...], (((1,), (1,)), ((), ())),
            preferred_element_type=jnp.float32)
        acc = acc + be_ref[...]
        pre_ref[j] = acc
        lane = lax.broadcasted_iota(jnp.int32, (r, 128), 1)
        m1 = jnp.full((r, 128), neg, jnp.float32)
        m2 = m1
        m3 = m1
        i1 = jnp.zeros((r, 128), jnp.int32)
        i2 = i1
        i3 = i1
        for s in range(nseg):
            v = acc[:, s * 128:(s + 1) * 128]
            iv = lane + (sae_blk * j + 128 * s)
            c1 = v > m1
            dm = jnp.where(c1, m1, v)
            di = jnp.where(c1, i1, iv)
            m1 = jnp.where(c1, v, m1)
            i1 = jnp.where(c1, iv, i1)
            c2 = dm > m2
            dm2 = jnp.where(c2, m2, dm)
            di2 = jnp.where(c2, i2, di)
            m2 = jnp.where(c2, dm, m2)
            i2 = jnp.where(c2, di, i2)
            c3 = dm2 > m3
            m3 = jnp.where(c3, dm2, m3)
            i3 = jnp.where(c3, di2, i3)
        cv_ref[3 * j] = m1
        cv_ref[3 * j + 1] = m2
        cv_ref[3 * j + 2] = m3
        ci_ref[3 * j] = i1
        ci_ref[3 * j + 1] = i2
        ci_ref[3 * j + 2] = i3

    @pl.when(j == nj - 1)
    def _topk():
        m_prev = jnp.full((r, 128), jnp.inf, jnp.float32)
        tops = []
        topis = []
        for _ in range(LANE_DEPTH):
            def plane_body(p, carry, m_prev=m_prev):
                m, im = carry
                v = cv_ref[p]
                iv = ci_ref[p]
                vm = jnp.where(v < m_prev, v, neg)
                take = vm > m
                return jnp.where(take, vm, m), jnp.where(take, iv, im)
            m_t, i_t = lax.fori_loop(
                0, 3 * nj, plane_body,
                (jnp.full((r, 128), neg, jnp.float32),
                 jnp.zeros((r, 128), jnp.int32)))
            tops.append(m_t)
            topis.append(i_t)
            m_prev = m_t
        keys = _f32_key(jnp.stack(tops, axis=0))

        def bis(_, carry):
            lo, hi = carry
            span = hi - lo
            mid = lo + (span >> jnp.uint32(1)) + (span & jnp.uint32(1))
            cnt = jnp.sum((keys >= mid[None, :, :]).astype(jnp.int32),
                          axis=(0, 2))[:, None]
            ge = cnt >= K
            return jnp.where(ge, mid, lo), jnp.where(ge, hi, mid - jnp.uint32(1))

        lo0 = jnp.zeros((r, 1), jnp.uint32)
        hi0 = jnp.full((r, 1), 0xFFFFFFFF, jnp.uint32)
        lo, _ = lax.fori_loop(0, 32, bis, (lo0, hi0))
        tval = _key_to_f32(lo)
        tval_ref[...] = tval
        nc = LANE_DEPTH * 128
        c640 = jnp.concatenate(tops, axis=1)
        i640 = jnp.concatenate(topis, axis=1)
        mask = c640 >= tval
        rowi = lax.broadcasted_iota(jnp.int32, (nc, nc), 0)
        coli = lax.broadcasted_iota(jnp.int32, (nc, nc), 1)
        lt = (rowi <= coli).astype(jnp.bfloat16)
        ranks = lax.dot_general(
            mask.astype(jnp.bfloat16), lt, (((1,), (0,)), ((), ())),
            preferred_element_type=jnp.float32).astype(jnp.int32) - 1
        vcols = []
        icols = []
        for t in range(K):
            sel = jnp.logical_and(mask, ranks == t)
            v_t = jnp.sum(jnp.where(sel, c640, jnp.float32(0.0)), axis=1,
                          keepdims=True)
            i_t = jnp.sum(jnp.where(sel, i640, 0), axis=1, keepdims=True)
            vcols.append(jnp.broadcast_to(v_t, (r, 16)))
            icols.append(i_t)
        cval_ref[...] = jnp.concatenate(vcols, axis=1)
        cidx_ref[:, 0:K] = jnp.concatenate(icols, axis=1)

    @pl.when(j >= nj)
    def _mask():
        jj = j - nj
        c = pre_ref[jj]
        lat_ref[...] = jnp.where(c >= tval_ref[...], c, jnp.float32(0.0))


def _tc_call(x, W_enc, b_enc2d, *, r, nj):
    b, d = x.shape
    s = W_enc.shape[0]
    sae_blk = s // nj
    grid = (b // r, 2 * nj)
    body = functools.partial(_tc_body, nj=nj)
    return pl.pallas_call(
        body,
        grid=grid,
        in_specs=[
            pl.BlockSpec((r, d), lambda i, j: (i, 0)),
            pl.BlockSpec((sae_blk, d), lambda i, j: (jnp.minimum(j, nj - 1), 0)),
            pl.BlockSpec((1, sae_blk), lambda i, j: (0, jnp.minimum(j, nj - 1))),
        ],
        out_specs=[
            pl.BlockSpec((r, sae_blk), lambda i, j: (i, jnp.maximum(j - nj, 0))),
            pl.BlockSpec((r, 16 * K), lambda i, j: (i, 0)),
            pl.BlockSpec((r, 128), lambda i, j: (i, 0)),
        ],
        out_shape=[
            jax.ShapeDtypeStruct((b, s), jnp.float32),
            jax.ShapeDtypeStruct((b, 16 * K), jnp.float32),
            jax.ShapeDtypeStruct((b, 128), jnp.int32),
        ],
        scratch_shapes=[
            pltpu.VMEM((nj, r, sae_blk), jnp.float32),
            pltpu.VMEM((3 * nj, r, 128), jnp.float32),
            pltpu.VMEM((3 * nj, r, 128), jnp.int32),
            pltpu.VMEM((r, 1), jnp.float32),
        ],
    )(x, W_enc, b_enc2d)


def _gather16(vec16, idxvec):
    dnums = lax.GatherDimensionNumbers(
        offset_dims=(), collapsed_slice_dims=(0,), start_index_map=(0,))
    return lax.gather(vec16, idxvec.astype(jnp.int32)[:, None], dnums, (1,),
                      mode=lax.GatherScatterMode.PROMISE_IN_BOUNDS)


def _splat(vec16, i):
    return _gather16(vec16, jnp.broadcast_to(i, (16,)))


def _prefix16(x):
    lanes = jnp.arange(16, dtype=jnp.int32)
    y = x
    for st in (1, 2, 4, 8):
        sh = _gather16(y, jnp.maximum(lanes - st, 0))
        y = y + jnp.where(lanes >= st, sh, 0)
    return y


def _sc_decode(cval, cidx, wdt):
    b = cval.shape[0]
    s, d = wdt.shape
    ndv = d // 16
    info = plsc.get_sparse_core_info()
    nc, ns = info.num_cores, info.num_subcores
    nw = nc * ns
    rows_per_w = b // nw
    mesh = plsc.VectorSubcoreMesh(core_axis_name="c", subcore_axis_name="s")

    @functools.partial(
        pl.kernel, mesh=mesh,
        out_type=jax.ShapeDtypeStruct((b, d), jnp.float32),
        scratch_types=[
            pltpu.VMEM((16 * K,), jnp.float32),
            pltpu.VMEM((128,), jnp.int32),
            pltpu.VMEM((K,), jnp.int32),
            pltpu.VMEM((K, d), jnp.float32),
            pltpu.VMEM((d,), jnp.float32),
            pltpu.SemaphoreType.DMA,
        ],
    )
    def k(cval_hbm, cidx_hbm, wdt_hbm, out_hbm,
          vals_v, idx_v, idx32_v, rows_v, rec_v, sem):
        wid = lax.axis_index("s") * nc + lax.axis_index("c")
        base = wid * rows_per_w

        def row_body(g, _):
            row = base + g
            pltpu.sync_copy(cval_hbm.at[row], vals_v)
            pltpu.sync_copy(cidx_hbm.at[row], idx_v)
            idx32_v[pl.ds(0, 16)] = idx_v[pl.ds(0, 16)]
            idx32_v[pl.ds(16, 16)] = idx_v[pl.ds(16, 16)]
            pltpu.async_copy(wdt_hbm.at[idx32_v], rows_v, sem).wait()

            def mac_body(kk, rec):
                vs = vals_v[pl.ds(kk * 16, 16)]
                return tuple(
                    rec[c] + vs * rows_v[kk, pl.ds(c * 16, 16)]
                    for c in range(ndv))

            rec = lax.fori_loop(
                0, K, mac_body,
                tuple(jnp.zeros((16,), jnp.float32) for _ in range(ndv)))
            for c in range(ndv):
                rec_v[pl.ds(c * 16, 16)] = rec[c]
            pltpu.sync_copy(rec_v, out_hbm.at[row])
            return 0

        lax.fori_loop(0, rows_per_w, row_body, 0)

    return k(cval, cidx, wdt)


def kernel(x, W_enc, b_enc, W_dec):
    b, d = x.shape
    s = W_enc.shape[0]
    lat, cval, cidx = _tc_call(x, W_enc, b_enc.reshape(1, s), r=256, nj=12)
    rec = _sc_decode(cval, cidx, W_dec.T.copy())
    aux = jnp.zeros((), jnp.float32)
    return (rec, lat, aux)
